# Initial kernel scaffold; baseline (speedup 1.0000x reference)
import jax
import jax.numpy as jnp
from jax.experimental import pallas as pl


def kernel(x, weight):
    raise NotImplementedError("write your pallas kernel here")


# speedup vs baseline: 2.4105x; 2.4105x over previous
import math

import jax
import jax.numpy as jnp
from jax import lax
from jax.experimental import pallas as pl
from jax.experimental.pallas import tpu as pltpu
from jax.experimental.pallas import tpu_sc as plsc

VOCAB_SIZE = 100000
D_MODEL = 128
BATCH = 4096
SEQ_LEN = 50
SCALE = math.sqrt(D_MODEL)

NC = 2
NS = 16
NW = NC * NS

TOTAL = BATCH * SEQ_LEN
PER_W = TOTAL // NW
CHUNK = 128
N_CHUNKS = PER_W // CHUNK
VECS = CHUNK * D_MODEL // 16


def _body(x_hbm, w_hbm, out_hbm, idx_v, rows_v, sem):
    wid = lax.axis_index("s") * NC + lax.axis_index("c")
    base = wid * PER_W
    pltpu.sync_copy(x_hbm.at[wid], idx_v)

    def chunk(c, carry):
        pltpu.async_copy(w_hbm.at[idx_v.at[c]], rows_v, sem).wait()

        def scale(i, carry2):
            r = i // (D_MODEL // 16)
            col = (i % (D_MODEL // 16)) * 16
            rows_v[r, pl.ds(col, 16)] = rows_v[r, pl.ds(col, 16)] * SCALE
            return carry2

        lax.fori_loop(0, VECS, scale, 0, unroll=8)
        pltpu.sync_copy(rows_v, out_hbm.at[pl.ds(base + c * CHUNK, CHUNK)])
        return carry

    lax.fori_loop(0, N_CHUNKS, chunk, 0)


@jax.jit
def kernel(x, weight):
    xf = x.reshape(NW, N_CHUNKS, CHUNK)
    mesh = plsc.VectorSubcoreMesh(
        core_axis_name="c", subcore_axis_name="s", num_cores=NC, num_subcores=NS
    )
    out = pl.kernel(
        _body,
        out_type=jax.ShapeDtypeStruct((TOTAL, D_MODEL), jnp.float32),
        mesh=mesh,
        scratch_types=[
            pltpu.VMEM((N_CHUNKS, CHUNK), jnp.int32),
            pltpu.VMEM((CHUNK, D_MODEL), jnp.float32),
            pltpu.SemaphoreType.DMA,
        ],
    )(xf, weight)
    return out.reshape(BATCH, SEQ_LEN, D_MODEL)


# speedup vs baseline: 2.5239x; 1.0470x over previous
import math

import jax
import jax.numpy as jnp
from jax import lax
from jax.experimental import pallas as pl
from jax.experimental.pallas import tpu as pltpu
from jax.experimental.pallas import tpu_sc as plsc

VOCAB_SIZE = 100000
D_MODEL = 128
BATCH = 4096
SEQ_LEN = 50
SCALE = math.sqrt(D_MODEL)

NC = 2
NS = 16
NW = NC * NS

TOTAL = BATCH * SEQ_LEN
PER_W = TOTAL // NW
CHUNK = 128
N_CHUNKS = PER_W // CHUNK
VECS = CHUNK * D_MODEL // 16


def _body(x_hbm, w_hbm, out_hbm, idx_v, rows_v, gsem, ssem):
    wid = lax.axis_index("s") * NC + lax.axis_index("c")
    base = wid * PER_W
    pltpu.sync_copy(x_hbm.at[wid], idx_v)

    def gather(c, buf):
        return pltpu.async_copy(w_hbm.at[idx_v.at[c]], rows_v.at[buf], gsem)

    def scatter(c, buf):
        return pltpu.async_copy(
            rows_v.at[buf], out_hbm.at[pl.ds(base + c * CHUNK, CHUNK)], ssem
        )

    def wait_gather(c, buf):
        pltpu.make_async_copy(
            w_hbm.at[idx_v.at[c]], rows_v.at[buf], gsem
        ).wait()

    def wait_scatter(c, buf):
        pltpu.make_async_copy(
            rows_v.at[buf], out_hbm.at[pl.ds(base + c * CHUNK, CHUNK)], ssem
        ).wait()

    gather(0, 0)

    @pl.loop(0, N_CHUNKS, step=2)
    def outer(c0):
        for k in range(2):
            c = c0 + k
            wait_gather(c, k)

            @plsc.parallel_loop(0, CHUNK)
            def scale_row(r):
                for j in range(D_MODEL // 16):
                    rows_v[k, r, pl.ds(j * 16, 16)] = (
                        rows_v[k, r, pl.ds(j * 16, 16)] * SCALE
                    )

            scatter(c, k)

            @pl.when(c > 0)
            def _():
                wait_scatter(c - 1, 1 - k)

            @pl.when(c < N_CHUNKS - 1)
            def _():
                gather(c + 1, 1 - k)

    wait_scatter(N_CHUNKS - 1, 1)


@jax.jit
def kernel(x, weight):
    xf = x.reshape(NW, N_CHUNKS, CHUNK)
    mesh = plsc.VectorSubcoreMesh(
        core_axis_name="c", subcore_axis_name="s", num_cores=NC, num_subcores=NS
    )
    out = pl.kernel(
        _body,
        out_type=jax.ShapeDtypeStruct((TOTAL, D_MODEL), jnp.float32),
        mesh=mesh,
        scratch_types=[
            pltpu.VMEM((N_CHUNKS, CHUNK), jnp.int32),
            pltpu.VMEM((2, CHUNK, D_MODEL), jnp.float32),
            pltpu.SemaphoreType.DMA,
            pltpu.SemaphoreType.DMA,
        ],
    )(xf, weight)
    return out.reshape(BATCH, SEQ_LEN, D_MODEL)


# speedup vs baseline: 2.9383x; 1.1642x over previous
import math

import jax
import jax.numpy as jnp
from jax import lax
from jax.experimental import pallas as pl
from jax.experimental.pallas import tpu as pltpu
from jax.experimental.pallas import tpu_sc as plsc

VOCAB_SIZE = 100000
D_MODEL = 128
BATCH = 4096
SEQ_LEN = 50
SCALE = math.sqrt(D_MODEL)

NC = 2
NS = 16
NW = NC * NS

TOTAL = BATCH * SEQ_LEN
PER_W = TOTAL // NW
CHUNK = 128
N_CHUNKS = PER_W // CHUNK
VECS = CHUNK * D_MODEL // 16
NBUF = 5


def _body(x_hbm, w_hbm, out_hbm, idx_v, rows_v, gsem, ssem):
    wid = lax.axis_index("s") * NC + lax.axis_index("c")
    base = wid * PER_W
    pltpu.sync_copy(x_hbm.at[wid], idx_v)

    def gather(c, buf):
        return pltpu.async_copy(w_hbm.at[idx_v.at[c]], rows_v.at[buf], gsem)

    def scatter(c, buf):
        return pltpu.async_copy(
            rows_v.at[buf], out_hbm.at[pl.ds(base + c * CHUNK, CHUNK)], ssem
        )

    def wait_gather(c, buf):
        pltpu.make_async_copy(
            w_hbm.at[idx_v.at[c]], rows_v.at[buf], gsem
        ).wait()

    def wait_scatter(c, buf):
        pltpu.make_async_copy(
            rows_v.at[buf], out_hbm.at[pl.ds(base + c * CHUNK, CHUNK)], ssem
        ).wait()

    for b in range(NBUF - 1):
        gather(b, b)

    @pl.loop(0, N_CHUNKS, step=NBUF)
    def outer(c0):
        for k in range(NBUF):
            c = c0 + k
            prev = (k - 1) % NBUF
            wait_gather(c, k)

            @pl.when(c > 0)
            def _():
                wait_scatter(c - 1, prev)

            @pl.when(c + NBUF - 1 < N_CHUNKS)
            def _():
                gather(c + NBUF - 1, prev)

            @plsc.parallel_loop(0, CHUNK, unroll=4)
            def scale_row(r):
                for j in range(D_MODEL // 16):
                    rows_v[k, r, pl.ds(j * 16, 16)] = (
                        rows_v[k, r, pl.ds(j * 16, 16)] * SCALE
                    )

            scatter(c, k)

    wait_scatter(N_CHUNKS - 1, (N_CHUNKS - 1) % NBUF)


@jax.jit
def kernel(x, weight):
    xf = x.reshape(NW, N_CHUNKS, CHUNK)
    mesh = plsc.VectorSubcoreMesh(
        core_axis_name="c", subcore_axis_name="s", num_cores=NC, num_subcores=NS
    )
    out = pl.kernel(
        _body,
        out_type=jax.ShapeDtypeStruct((TOTAL, D_MODEL), jnp.float32),
        mesh=mesh,
        scratch_types=[
            pltpu.VMEM((N_CHUNKS, CHUNK), jnp.int32),
            pltpu.VMEM((NBUF, CHUNK, D_MODEL), jnp.float32),
            pltpu.SemaphoreType.DMA,
            pltpu.SemaphoreType.DMA,
        ],
    )(xf, weight)
    return out.reshape(BATCH, SEQ_LEN, D_MODEL)


# speedup vs baseline: 5.2582x; 1.7896x over previous
import math

import jax
import jax.numpy as jnp
from jax import lax
from jax.experimental import pallas as pl
from jax.experimental.pallas import tpu as pltpu
from jax.experimental.pallas import tpu_sc as plsc

VOCAB_SIZE = 100000
D_MODEL = 128
BATCH = 4096
SEQ_LEN = 50
SCALE = math.sqrt(D_MODEL)

NC = 2
NS = 16
NW = NC * NS

B_PER_W = BATCH // NW
CHUNK_B = 2
CHUNK = CHUNK_B * SEQ_LEN
N_CHUNKS = B_PER_W // CHUNK_B
NBUF = 4


def _body(x_hbm, w_hbm, out_hbm, idx_v, rows_v, gsem, ssem):
    wid = lax.axis_index("s") * NC + lax.axis_index("c")
    b_base = wid * B_PER_W
    pltpu.sync_copy(x_hbm.at[wid], idx_v)

    def gather(c, buf):
        return pltpu.async_copy(w_hbm.at[idx_v.at[c]], rows_v.at[buf], gsem)

    def wait_gather(c, buf):
        pltpu.make_async_copy(
            w_hbm.at[idx_v.at[c]], rows_v.at[buf], gsem
        ).wait()

    def scatter(c, buf, wait):
        for i in range(CHUNK_B):
            src = rows_v.at[buf, pl.ds(i * SEQ_LEN, SEQ_LEN)]
            dst = out_hbm.at[b_base + c * CHUNK_B + i]
            if wait:
                pltpu.make_async_copy(src, dst, ssem).wait()
            else:
                pltpu.async_copy(src, dst, ssem)

    for b in range(NBUF - 1):
        gather(b, b)

    @pl.loop(0, N_CHUNKS, step=NBUF)
    def outer(c0):
        for k in range(NBUF):
            c = c0 + k
            prev = (k - 1) % NBUF
            wait_gather(c, k)

            @pl.when(c > 0)
            def _():
                scatter(c - 1, prev, wait=True)

            @pl.when(c + NBUF - 1 < N_CHUNKS)
            def _():
                gather(c + NBUF - 1, prev)

            @plsc.parallel_loop(0, CHUNK, unroll=4)
            def scale_row(r):
                for j in range(D_MODEL // 16):
                    rows_v[k, r, pl.ds(j * 16, 16)] = (
                        rows_v[k, r, pl.ds(j * 16, 16)] * SCALE
                    )

            scatter(c, k, wait=False)

    scatter(N_CHUNKS - 1, (N_CHUNKS - 1) % NBUF, wait=True)


@jax.jit
def kernel(x, weight):
    xf = x.reshape(NW, N_CHUNKS, CHUNK)
    mesh = plsc.VectorSubcoreMesh(
        core_axis_name="c", subcore_axis_name="s", num_cores=NC, num_subcores=NS
    )
    return pl.kernel(
        _body,
        out_type=jax.ShapeDtypeStruct((BATCH, SEQ_LEN, D_MODEL), jnp.float32),
        mesh=mesh,
        scratch_types=[
            pltpu.VMEM((N_CHUNKS, CHUNK), jnp.int32),
            pltpu.VMEM((NBUF, CHUNK, D_MODEL), jnp.float32),
            pltpu.SemaphoreType.DMA,
            pltpu.SemaphoreType.DMA,
        ],
    )(xf, weight)
